# Initial kernel scaffold; baseline (speedup 1.0000x reference)
import jax
import jax.numpy as jnp
from jax.experimental import pallas as pl


def kernel(x, edge_index, W1, att_src1, att_dst1, b1, W2, att_src2, att_dst2, b2):
    raise NotImplementedError("write your pallas kernel here")


# speedup vs baseline: 37.2065x; 37.2065x over previous
import functools

import jax
import jax.numpy as jnp
from jax import lax
from jax.experimental import pallas as pl
from jax.experimental.pallas import tpu as pltpu
from jax.experimental.pallas import tpu_sc as plsc

N = 10000
E = 320000
E2 = E + N
FIN = 128
H1 = 8
HID = 16
F1 = H1 * HID

NC = 2
NS = 16
NW = NC * NS
L = 16

KC = 128
NCHUNK = 81
P = KC * NCHUNK
E2P = P * NW
TROWS = 632

@functools.cache
def _mesh():
    return plsc.VectorSubcoreMesh(
        core_axis_name="c", subcore_axis_name="s",
        num_cores=NC, num_subcores=NS)


_SC_PARAMS = pltpu.CompilerParams(
    use_tc_tiling_on_sc=False, needs_layout_passes=False)


def _feat_body(x_ref, w_ref, as_ref, ad_ref, h_ref, a_s_ref, a_d_ref,
               ms_ref, md_ref):
    i = pl.program_id(0)
    h = jnp.dot(x_ref[...], w_ref[...], preferred_element_type=jnp.float32)
    h_ref[...] = h
    a_s = jnp.dot(h, as_ref[...], preferred_element_type=jnp.float32)
    a_d = jnp.dot(h, ad_ref[...], preferred_element_type=jnp.float32)
    a_s_ref[...] = a_s
    a_d_ref[...] = a_d
    pms = jnp.broadcast_to(jnp.max(a_s, axis=0, keepdims=True), (8, 8))
    pmd = jnp.broadcast_to(jnp.max(a_d, axis=0, keepdims=True), (8, 8))

    @pl.when(i == 0)
    def _():
        ms_ref[...] = pms
        md_ref[...] = pmd

    @pl.when(i > 0)
    def _():
        ms_ref[...] = jnp.maximum(ms_ref[...], pms)
        md_ref[...] = jnp.maximum(md_ref[...], pmd)


def _feat1(x, W1, A1s, A1d):
    bn = 1000
    grid = N // bn
    return pl.pallas_call(
        _feat_body,
        grid=(grid,),
        in_specs=[
            pl.BlockSpec((bn, FIN), lambda i: (i, 0)),
            pl.BlockSpec((FIN, F1), lambda i: (0, 0)),
            pl.BlockSpec((F1, 8), lambda i: (0, 0)),
            pl.BlockSpec((F1, 8), lambda i: (0, 0)),
        ],
        out_specs=[
            pl.BlockSpec((bn, F1), lambda i: (i, 0)),
            pl.BlockSpec((bn, 8), lambda i: (i, 0)),
            pl.BlockSpec((bn, 8), lambda i: (i, 0)),
            pl.BlockSpec((8, 8), lambda i: (0, 0)),
            pl.BlockSpec((8, 8), lambda i: (0, 0)),
        ],
        out_shape=[
            jax.ShapeDtypeStruct((N, F1), jnp.float32),
            jax.ShapeDtypeStruct((N, 8), jnp.float32),
            jax.ShapeDtypeStruct((N, 8), jnp.float32),
            jax.ShapeDtypeStruct((8, 8), jnp.float32),
            jax.ShapeDtypeStruct((8, 8), jnp.float32),
        ],
    )(x, W1, A1s, A1d)


def _feat2_body(p_ref, b_ref, w_ref, as_ref, ad_ref, h2_ref, a_s_ref,
                a_d_ref, ms_ref, md_ref):
    i = pl.program_id(0)
    h = jax.nn.relu(p_ref[0] + p_ref[1] + b_ref[...])
    h2 = jnp.dot(h, w_ref[...], preferred_element_type=jnp.float32)
    h2_ref[...] = h2
    a_s = jnp.dot(h2, as_ref[...], preferred_element_type=jnp.float32)
    a_d = jnp.dot(h2, ad_ref[...], preferred_element_type=jnp.float32)
    a_s_ref[...] = a_s
    a_d_ref[...] = a_d
    pms = jnp.broadcast_to(jnp.max(a_s, axis=0, keepdims=True), (8, 8))
    pmd = jnp.broadcast_to(jnp.max(a_d, axis=0, keepdims=True), (8, 8))

    @pl.when(i == 0)
    def _():
        ms_ref[...] = pms
        md_ref[...] = pmd

    @pl.when(i > 0)
    def _():
        ms_ref[...] = jnp.maximum(ms_ref[...], pms)
        md_ref[...] = jnp.maximum(md_ref[...], pmd)


def _feat2(parts, b1r, W2, A2s, A2d):
    bn = 1000
    grid = N // bn
    return pl.pallas_call(
        _feat2_body,
        grid=(grid,),
        in_specs=[
            pl.BlockSpec((2, bn, F1), lambda i: (0, i, 0)),
            pl.BlockSpec((1, F1), lambda i: (0, 0)),
            pl.BlockSpec((F1, HID), lambda i: (0, 0)),
            pl.BlockSpec((HID, 8), lambda i: (0, 0)),
            pl.BlockSpec((HID, 8), lambda i: (0, 0)),
        ],
        out_specs=[
            pl.BlockSpec((bn, HID), lambda i: (i, 0)),
            pl.BlockSpec((bn, 8), lambda i: (i, 0)),
            pl.BlockSpec((bn, 8), lambda i: (i, 0)),
            pl.BlockSpec((8, 8), lambda i: (0, 0)),
            pl.BlockSpec((8, 8), lambda i: (0, 0)),
        ],
        out_shape=[
            jax.ShapeDtypeStruct((N, HID), jnp.float32),
            jax.ShapeDtypeStruct((N, 8), jnp.float32),
            jax.ShapeDtypeStruct((N, 8), jnp.float32),
            jax.ShapeDtypeStruct((8, 8), jnp.float32),
            jax.ShapeDtypeStruct((8, 8), jnp.float32),
        ],
    )(parts, b1r, W2, A2s, A2d)


def _inv_body(d_ref, inv_ref):
    inv_ref[...] = 1.0 / (d_ref[0] + d_ref[1] + 1e-16)


def _inv_denom(den_parts):
    bn = 1000
    return pl.pallas_call(
        _inv_body,
        grid=(N // bn,),
        in_specs=[pl.BlockSpec((2, bn, 8), lambda i: (0, i, 0))],
        out_specs=pl.BlockSpec((bn, 8), lambda i: (i, 0)),
        out_shape=jax.ShapeDtypeStruct((N, 8), jnp.float32),
    )(den_parts)


def _final_body(p_ref, b_ref, o_ref):
    o_ref[...] = p_ref[0] + p_ref[1] + b_ref[...]


def _final(parts, b2r):
    bn = 1000
    return pl.pallas_call(
        _final_body,
        grid=(N // bn,),
        in_specs=[
            pl.BlockSpec((2, bn, HID), lambda i: (0, i, 0)),
            pl.BlockSpec((1, HID), lambda i: (0, 0)),
        ],
        out_specs=pl.BlockSpec((bn, HID), lambda i: (i, 0)),
        out_shape=jax.ShapeDtypeStruct((N, HID), jnp.float32),
    )(parts, b2r)


def _lane_rc(k):
    i = lax.iota(jnp.int32, L)
    erow = 2 * k + lax.shift_right_logical(i, 3)
    col = lax.bitwise_and(i, 7)
    return erow, col


def _edge_softmax_body(as_h, ad_h, src_h, dst_h, mvec_h, zer_h,
                       ex_h, den_h,
                       sidx, didx, asv, adv, exv, mv, sem, den_acc):
    c = lax.axis_index("c")
    s = lax.axis_index("s")
    wid = c * NS + s
    o = jnp.minimum(s * TROWS, N - TROWS)
    pltpu.sync_copy(zer_h, den_acc.at[pl.ds(o, TROWS)])
    pltpu.sync_copy(src_h.at[wid], sidx)
    pltpu.sync_copy(dst_h.at[wid], didx)
    pltpu.sync_copy(mvec_h, mv)
    plsc.subcore_barrier()
    base_w = wid * P

    def chunk(g, carry):
        cp1 = pltpu.async_copy(as_h.at[sidx.at[g]], asv, sem)
        cp2 = pltpu.async_copy(ad_h.at[didx.at[g]], adv, sem)
        cp1.wait()
        cp2.wait()
        mvv = mv[...]
        base = base_w + g * KC

        def lanes(k, carry2):
            erow, col = _lane_rc(k)
            a = plsc.load_gather(asv, [erow, col])
            b = plsc.load_gather(adv, [erow, col])
            z = a + b
            alpha = jnp.maximum(z, 0.2 * z)
            ex = jnp.exp(alpha - mvv)
            gidx = base + erow
            ex = jnp.where(gidx < E2, ex, 0.0)
            plsc.store_scatter(exv, [erow, col], ex)
            return carry2

        lax.fori_loop(0, KC // 2, lanes, 0)
        pltpu.sync_copy(exv, ex_h.at[wid, g])
        pltpu.sync_copy(exv, den_acc.at[didx.at[g]], add=True)
        return carry

    lax.fori_loop(0, NCHUNK, chunk, 0)
    plsc.subcore_barrier()
    pltpu.sync_copy(den_acc.at[pl.ds(o, TROWS)],
                    den_h.at[c, pl.ds(o, TROWS)])


def _edge_softmax(a_s, a_d, src3, dst3, mvec, zer8):
    return pl.kernel(
        _edge_softmax_body,
        out_type=[
            jax.ShapeDtypeStruct((NW, NCHUNK, KC, 8), jnp.float32),
            jax.ShapeDtypeStruct((NC, N, 8), jnp.float32),
        ],
        mesh=_mesh(),
        compiler_params=_SC_PARAMS,
        scratch_types=[
            pltpu.VMEM((NCHUNK, KC), jnp.int32),
            pltpu.VMEM((NCHUNK, KC), jnp.int32),
            pltpu.VMEM((KC, 8), jnp.float32),
            pltpu.VMEM((KC, 8), jnp.float32),
            pltpu.VMEM((KC, 8), jnp.float32),
            pltpu.VMEM((L,), jnp.float32),
            pltpu.SemaphoreType.DMA,
            pltpu.VMEM_SHARED((N, 8), jnp.float32),
        ],
    )(a_s, a_d, src3, dst3, mvec, zer8)


def _make_scatter_body(R):
    RW = L * R

    def body(h_h, ex_h, inv_h, src_h, dst_h, zer_h, out_h,
             sidx, didx, hv, exv, invv, coefv, sem, acc):
        c = lax.axis_index("c")
        s = lax.axis_index("s")
        wid = c * NS + s
        o = jnp.minimum(s * TROWS, N - TROWS)
        pltpu.sync_copy(zer_h, acc.at[pl.ds(o, TROWS)])
        pltpu.sync_copy(src_h.at[wid], sidx)
        pltpu.sync_copy(dst_h.at[wid], didx)
        plsc.subcore_barrier()

        def chunk(g, carry):
            cp0 = pltpu.async_copy(ex_h.at[wid, g], exv, sem)
            cp1 = pltpu.async_copy(inv_h.at[didx.at[g]], invv, sem)
            cp2 = pltpu.async_copy(h_h.at[sidx.at[g]], hv, sem)
            cp0.wait()
            cp1.wait()
            cp2.wait()

            def lanes(k, carry2):
                erow, col = _lane_rc(k)
                e = plsc.load_gather(exv, [erow, col])
                iv = plsc.load_gather(invv, [erow, col])
                plsc.store_scatter(coefv, [erow, col], e * iv)
                return carry2

            lax.fori_loop(0, KC // 2, lanes, 0)

            def edge(e, carry2):
                esp = jnp.full((L,), 0, jnp.int32) + e
                for j in range(R):
                    cv = plsc.load_gather(
                        coefv, [esp, jnp.full((L,), j, jnp.int32)])
                    hrow = hv[e, pl.ds(j * L, L)]
                    hv[e, pl.ds(j * L, L)] = hrow * cv
                return carry2

            lax.fori_loop(0, KC, edge, 0)
            pltpu.sync_copy(hv, acc.at[didx.at[g]], add=True)
            return carry

        lax.fori_loop(0, NCHUNK, chunk, 0)
        plsc.subcore_barrier()
        pltpu.sync_copy(acc.at[pl.ds(o, TROWS)],
                        out_h.at[c, pl.ds(o, TROWS)])

    return body, RW


def _weighted_scatter(R, h, ex4, inv, src3, dst3, zer):
    body, RW = _make_scatter_body(R)
    return pl.kernel(
        body,
        out_type=jax.ShapeDtypeStruct((NC, N, RW), jnp.float32),
        mesh=_mesh(),
        compiler_params=_SC_PARAMS,
        scratch_types=[
            pltpu.VMEM((NCHUNK, KC), jnp.int32),
            pltpu.VMEM((NCHUNK, KC), jnp.int32),
            pltpu.VMEM((KC, RW), jnp.float32),
            pltpu.VMEM((KC, 8), jnp.float32),
            pltpu.VMEM((KC, 8), jnp.float32),
            pltpu.VMEM((KC, 8), jnp.float32),
            pltpu.SemaphoreType.DMA,
            pltpu.VMEM_SHARED((N, RW), jnp.float32),
        ],
    )(h, ex4, inv, src3, dst3, zer)


def kernel(x, edge_index, W1, att_src1, att_dst1, b1, W2, att_src2,
           att_dst2, b2):
    loop = jnp.arange(N, dtype=edge_index.dtype)
    src = jnp.concatenate([edge_index[0], loop,
                           jnp.zeros((E2P - E2,), edge_index.dtype)])
    dst = jnp.concatenate([edge_index[1], loop,
                           jnp.zeros((E2P - E2,), edge_index.dtype)])
    src3 = src.reshape(NW, NCHUNK, KC)
    dst3 = dst.reshape(NW, NCHUNK, KC)

    eye8 = jnp.eye(H1, dtype=jnp.float32)
    A1s = (att_src1[:, :, None] * eye8[:, None, :]).reshape(F1, H1)
    A1d = (att_dst1[:, :, None] * eye8[:, None, :]).reshape(F1, H1)
    A2s = jnp.tile(att_src2.reshape(HID, 1), (1, 8))
    A2d = jnp.tile(att_dst2.reshape(HID, 1), (1, 8))

    zer8 = jnp.zeros((TROWS, 8), jnp.float32)
    zer128 = jnp.zeros((TROWS, F1), jnp.float32)
    zer16 = jnp.zeros((TROWS, HID), jnp.float32)

    h1, a_s1, a_d1, ms1, md1 = _feat1(x, W1, A1s, A1d)
    z1 = ms1[0] + md1[0]
    m1 = jnp.maximum(z1, 0.2 * z1)
    m1vec = jnp.tile(m1, 2)
    ex1, den1 = _edge_softmax(a_s1, a_d1, src3, dst3, m1vec, zer8)
    inv1 = _inv_denom(den1)
    out1p = _weighted_scatter(H1, h1, ex1, inv1, src3, dst3, zer128)

    h2, a_s2, a_d2, ms2, md2 = _feat2(out1p, b1.reshape(1, F1), W2, A2s, A2d)
    z2 = ms2[0] + md2[0]
    m2 = jnp.maximum(z2, 0.2 * z2)
    m2vec = jnp.tile(m2, 2)
    ex2, den2 = _edge_softmax(a_s2, a_d2, src3, dst3, m2vec, zer8)
    inv2 = _inv_denom(den2)
    out2p = _weighted_scatter(1, h2, ex2, inv2, src3, dst3, zer16)

    return _final(out2p, b2.reshape(1, HID))


# speedup vs baseline: 43.7861x; 1.1768x over previous
import functools

import jax
import jax.numpy as jnp
from jax import lax
from jax.experimental import pallas as pl
from jax.experimental.pallas import tpu as pltpu
from jax.experimental.pallas import tpu_sc as plsc

N = 10000
E = 320000
E2 = E + N
FIN = 128
H1 = 8
HID = 16
F1 = H1 * HID

NC = 2
NS = 16
NW = NC * NS
L = 16

KC = 128
NCHUNK = 81
P = KC * NCHUNK
E2P = P * NW
TROWS = 632


@functools.cache
def _mesh():
    return plsc.VectorSubcoreMesh(
        core_axis_name="c", subcore_axis_name="s",
        num_cores=NC, num_subcores=NS)


_SC_PARAMS = pltpu.CompilerParams(
    use_tc_tiling_on_sc=False, needs_layout_passes=False)


def _feat_body(x_ref, w_ref, as_ref, ad_ref, h_ref, a_s_ref, a_d_ref,
               ms_ref, md_ref):
    i = pl.program_id(0)
    h = jnp.dot(x_ref[...], w_ref[...], preferred_element_type=jnp.float32)
    h_ref[...] = h
    a_s = jnp.dot(h, as_ref[...], preferred_element_type=jnp.float32)
    a_d = jnp.dot(h, ad_ref[...], preferred_element_type=jnp.float32)
    a_s_ref[...] = a_s
    a_d_ref[...] = a_d
    pms = jnp.broadcast_to(jnp.max(a_s, axis=0, keepdims=True), (8, 8))
    pmd = jnp.broadcast_to(jnp.max(a_d, axis=0, keepdims=True), (8, 8))

    @pl.when(i == 0)
    def _():
        ms_ref[...] = pms
        md_ref[...] = pmd

    @pl.when(i > 0)
    def _():
        ms_ref[...] = jnp.maximum(ms_ref[...], pms)
        md_ref[...] = jnp.maximum(md_ref[...], pmd)


def _feat1(x, W1, A1s, A1d):
    bn = 1000
    grid = N // bn
    return pl.pallas_call(
        _feat_body,
        grid=(grid,),
        in_specs=[
            pl.BlockSpec((bn, FIN), lambda i: (i, 0)),
            pl.BlockSpec((FIN, F1), lambda i: (0, 0)),
            pl.BlockSpec((F1, 8), lambda i: (0, 0)),
            pl.BlockSpec((F1, 8), lambda i: (0, 0)),
        ],
        out_specs=[
            pl.BlockSpec((bn, F1), lambda i: (i, 0)),
            pl.BlockSpec((bn, 8), lambda i: (i, 0)),
            pl.BlockSpec((bn, 8), lambda i: (i, 0)),
            pl.BlockSpec((8, 8), lambda i: (0, 0)),
            pl.BlockSpec((8, 8), lambda i: (0, 0)),
        ],
        out_shape=[
            jax.ShapeDtypeStruct((N, F1), jnp.float32),
            jax.ShapeDtypeStruct((N, 8), jnp.float32),
            jax.ShapeDtypeStruct((N, 8), jnp.float32),
            jax.ShapeDtypeStruct((8, 8), jnp.float32),
            jax.ShapeDtypeStruct((8, 8), jnp.float32),
        ],
    )(x, W1, A1s, A1d)


def _combine1_body(p_ref, d_ref, b_ref, w_ref, as_ref, ad_ref,
                   h2_ref, a_s_ref, a_d_ref, ms_ref, md_ref):
    i = pl.program_id(0)
    bn = p_ref.shape[1]
    inv = 1.0 / (d_ref[0] + d_ref[1] + 1e-16)
    invx = jnp.broadcast_to(inv[:, :, None], (bn, 8, HID)).reshape(bn, F1)
    h = jax.nn.relu((p_ref[0] + p_ref[1]) * invx + b_ref[...])
    h2 = jnp.dot(h, w_ref[...], preferred_element_type=jnp.float32)
    h2_ref[...] = h2
    a_s = jnp.dot(h2, as_ref[...], preferred_element_type=jnp.float32)
    a_d = jnp.dot(h2, ad_ref[...], preferred_element_type=jnp.float32)
    a_s_ref[...] = a_s
    a_d_ref[...] = a_d
    pms = jnp.broadcast_to(jnp.max(a_s, axis=0, keepdims=True), (8, 8))
    pmd = jnp.broadcast_to(jnp.max(a_d, axis=0, keepdims=True), (8, 8))

    @pl.when(i == 0)
    def _():
        ms_ref[...] = pms
        md_ref[...] = pmd

    @pl.when(i > 0)
    def _():
        ms_ref[...] = jnp.maximum(ms_ref[...], pms)
        md_ref[...] = jnp.maximum(md_ref[...], pmd)


def _combine1(parts, den, b1r, W2, A2s, A2d):
    bn = 1000
    grid = N // bn
    return pl.pallas_call(
        _combine1_body,
        grid=(grid,),
        in_specs=[
            pl.BlockSpec((2, bn, F1), lambda i: (0, i, 0)),
            pl.BlockSpec((2, bn, 8), lambda i: (0, i, 0)),
            pl.BlockSpec((1, F1), lambda i: (0, 0)),
            pl.BlockSpec((F1, HID), lambda i: (0, 0)),
            pl.BlockSpec((HID, 8), lambda i: (0, 0)),
            pl.BlockSpec((HID, 8), lambda i: (0, 0)),
        ],
        out_specs=[
            pl.BlockSpec((bn, HID), lambda i: (i, 0)),
            pl.BlockSpec((bn, 8), lambda i: (i, 0)),
            pl.BlockSpec((bn, 8), lambda i: (i, 0)),
            pl.BlockSpec((8, 8), lambda i: (0, 0)),
            pl.BlockSpec((8, 8), lambda i: (0, 0)),
        ],
        out_shape=[
            jax.ShapeDtypeStruct((N, HID), jnp.float32),
            jax.ShapeDtypeStruct((N, 8), jnp.float32),
            jax.ShapeDtypeStruct((N, 8), jnp.float32),
            jax.ShapeDtypeStruct((8, 8), jnp.float32),
            jax.ShapeDtypeStruct((8, 8), jnp.float32),
        ],
    )(parts, den, b1r, W2, A2s, A2d)


def _final_body(p_ref, d_ref, b_ref, o_ref):
    bn = p_ref.shape[1]
    inv = 1.0 / (d_ref[0, :, 0:1] + d_ref[1, :, 0:1] + 1e-16)
    o_ref[...] = (p_ref[0] + p_ref[1]) * inv + b_ref[...]


def _final(parts, den, b2r):
    bn = 1000
    return pl.pallas_call(
        _final_body,
        grid=(N // bn,),
        in_specs=[
            pl.BlockSpec((2, bn, HID), lambda i: (0, i, 0)),
            pl.BlockSpec((2, bn, 8), lambda i: (0, i, 0)),
            pl.BlockSpec((1, HID), lambda i: (0, 0)),
        ],
        out_specs=pl.BlockSpec((bn, HID), lambda i: (i, 0)),
        out_shape=jax.ShapeDtypeStruct((N, HID), jnp.float32),
    )(parts, den, b2r)


def _lane_rc(k):
    i = lax.iota(jnp.int32, L)
    erow = 2 * k + lax.shift_right_logical(i, 3)
    col = lax.bitwise_and(i, 7)
    return erow, col


def _make_edge_body(R):
    RW = L * R

    def body(as_h, ad_h, h_h, src_h, dst_h, mvec_h, zden_h, zout_h,
             den_h, out_h,
             sidx, didx, asv, adv, exv, hv, mv, sem, den_acc, out_acc):
        c = lax.axis_index("c")
        s = lax.axis_index("s")
        wid = c * NS + s
        o = jnp.minimum(s * TROWS, N - TROWS)
        pltpu.sync_copy(zden_h, den_acc.at[pl.ds(o, TROWS)])
        pltpu.sync_copy(zout_h, out_acc.at[pl.ds(o, TROWS)])
        pltpu.sync_copy(src_h.at[wid], sidx)
        pltpu.sync_copy(dst_h.at[wid], didx)
        pltpu.sync_copy(mvec_h, mv)
        plsc.subcore_barrier()
        base_w = wid * P

        def chunk(g, carry):
            cp0 = pltpu.async_copy(as_h.at[sidx.at[g]], asv, sem)
            cp1 = pltpu.async_copy(ad_h.at[didx.at[g]], adv, sem)
            cp2 = pltpu.async_copy(h_h.at[sidx.at[g]], hv, sem)
            cp0.wait()
            cp1.wait()
            cp2.wait()
            mvv = mv[...]
            base = base_w + g * KC

            def lanes(k, carry2):
                erow, col = _lane_rc(k)
                a = plsc.load_gather(asv, [erow, col])
                b = plsc.load_gather(adv, [erow, col])
                z = a + b
                alpha = jnp.maximum(z, 0.2 * z)
                ex = jnp.exp(alpha - mvv)
                gidx = base + erow
                ex = jnp.where(gidx < E2, ex, 0.0)
                plsc.store_scatter(exv, [erow, col], ex)
                return carry2

            lax.fori_loop(0, KC // 2, lanes, 0)
            pltpu.sync_copy(exv, den_acc.at[didx.at[g]], add=True)

            def edge(e, carry2):
                esp = jnp.full((L,), 0, jnp.int32) + e
                for j in range(R):
                    cv = plsc.load_gather(
                        exv, [esp, jnp.full((L,), j, jnp.int32)])
                    hrow = hv[e, pl.ds(j * L, L)]
                    hv[e, pl.ds(j * L, L)] = hrow * cv
                return carry2

            lax.fori_loop(0, KC, edge, 0)
            pltpu.sync_copy(hv, out_acc.at[didx.at[g]], add=True)
            return carry

        lax.fori_loop(0, NCHUNK, chunk, 0)
        plsc.subcore_barrier()
        pltpu.sync_copy(den_acc.at[pl.ds(o, TROWS)],
                        den_h.at[c, pl.ds(o, TROWS)])
        pltpu.sync_copy(out_acc.at[pl.ds(o, TROWS)],
                        out_h.at[c, pl.ds(o, TROWS)])

    return body, RW


def _edge_sweep(R, a_s, a_d, h, src3, dst3, mvec, zden, zout):
    body, RW = _make_edge_body(R)
    return pl.kernel(
        body,
        out_type=[
            jax.ShapeDtypeStruct((NC, N, 8), jnp.float32),
            jax.ShapeDtypeStruct((NC, N, RW), jnp.float32),
        ],
        mesh=_mesh(),
        compiler_params=_SC_PARAMS,
        scratch_types=[
            pltpu.VMEM((NCHUNK, KC), jnp.int32),
            pltpu.VMEM((NCHUNK, KC), jnp.int32),
            pltpu.VMEM((KC, 8), jnp.float32),
            pltpu.VMEM((KC, 8), jnp.float32),
            pltpu.VMEM((KC, 8), jnp.float32),
            pltpu.VMEM((KC, RW), jnp.float32),
            pltpu.VMEM((L,), jnp.float32),
            pltpu.SemaphoreType.DMA,
            pltpu.VMEM_SHARED((N, 8), jnp.float32),
            pltpu.VMEM_SHARED((N, RW), jnp.float32),
        ],
    )(a_s, a_d, h, src3, dst3, mvec, zden, zout)


def kernel(x, edge_index, W1, att_src1, att_dst1, b1, W2, att_src2,
           att_dst2, b2):
    loop = jnp.arange(N, dtype=edge_index.dtype)
    src = jnp.concatenate([edge_index[0], loop,
                           jnp.zeros((E2P - E2,), edge_index.dtype)])
    dst = jnp.concatenate([edge_index[1], loop,
                           jnp.zeros((E2P - E2,), edge_index.dtype)])
    src3 = src.reshape(NW, NCHUNK, KC)
    dst3 = dst.reshape(NW, NCHUNK, KC)

    eye8 = jnp.eye(H1, dtype=jnp.float32)
    A1s = (att_src1[:, :, None] * eye8[:, None, :]).reshape(F1, H1)
    A1d = (att_dst1[:, :, None] * eye8[:, None, :]).reshape(F1, H1)
    A2s = jnp.tile(att_src2.reshape(HID, 1), (1, 8))
    A2d = jnp.tile(att_dst2.reshape(HID, 1), (1, 8))

    zden = jnp.zeros((TROWS, 8), jnp.float32)
    zout1 = jnp.zeros((TROWS, F1), jnp.float32)
    zout2 = jnp.zeros((TROWS, HID), jnp.float32)

    h1, a_s1, a_d1, ms1, md1 = _feat1(x, W1, A1s, A1d)
    z1 = ms1[0] + md1[0]
    m1vec = jnp.tile(jnp.maximum(z1, 0.2 * z1), 2)
    den1, out1p = _edge_sweep(H1, a_s1, a_d1, h1, src3, dst3, m1vec,
                              zden, zout1)

    h2, a_s2, a_d2, ms2, md2 = _combine1(out1p, den1, b1.reshape(1, F1),
                                         W2, A2s, A2d)
    z2 = ms2[0] + md2[0]
    m2vec = jnp.tile(jnp.maximum(z2, 0.2 * z2), 2)
    den2, out2p = _edge_sweep(1, a_s2, a_d2, h2, src3, dst3, m2vec,
                              zden, zout2)

    return _final(out2p, den2, b2.reshape(1, HID))


# speedup vs baseline: 48.3495x; 1.1042x over previous
import functools

import jax
import jax.numpy as jnp
from jax import lax
from jax.experimental import pallas as pl
from jax.experimental.pallas import tpu as pltpu
from jax.experimental.pallas import tpu_sc as plsc

N = 10000
E = 320000
E2 = E + N
FIN = 128
H1 = 8
HID = 16
F1 = H1 * HID
FH = F1 // 2
HXW = FH + 8

NC = 2
NS = 16
NW = NC * NS
L = 16

KC = 128
NCHUNK = 81
P = KC * NCHUNK
NCHUNK1 = 162
P1 = KC * NCHUNK1
E2P = P * NW
NBUF = 3
TROWS = 632


@functools.cache
def _mesh():
    return plsc.VectorSubcoreMesh(
        core_axis_name="c", subcore_axis_name="s",
        num_cores=NC, num_subcores=NS)


_SC_PARAMS = pltpu.CompilerParams(
    use_tc_tiling_on_sc=False, needs_layout_passes=False)


def _feat_body(x_ref, w_ref, as_ref, ad_ref, hx_ref, a_d_ref, ms_ref,
               md_ref):
    i = pl.program_id(0)
    bn = x_ref.shape[0]
    h = jnp.dot(x_ref[...], w_ref[...], preferred_element_type=jnp.float32)
    a_s = jnp.dot(h, as_ref[...], preferred_element_type=jnp.float32)
    a_d = jnp.dot(h, ad_ref[...], preferred_element_type=jnp.float32)
    a_d_ref[...] = a_d
    pad = jnp.zeros((bn, 4), jnp.float32)
    hx_ref[0] = jnp.concatenate([h[:, :FH], a_s[:, :4], pad], axis=1)
    hx_ref[1] = jnp.concatenate([h[:, FH:], a_s[:, 4:], pad], axis=1)
    pms = jnp.broadcast_to(jnp.max(a_s, axis=0, keepdims=True), (8, 8))
    pmd = jnp.broadcast_to(jnp.max(a_d, axis=0, keepdims=True), (8, 8))

    @pl.when(i == 0)
    def _():
        ms_ref[...] = pms
        md_ref[...] = pmd

    @pl.when(i > 0)
    def _():
        ms_ref[...] = jnp.maximum(ms_ref[...], pms)
        md_ref[...] = jnp.maximum(md_ref[...], pmd)


def _feat1(x, W1, A1s, A1d):
    bn = 1000
    grid = N // bn
    return pl.pallas_call(
        _feat_body,
        grid=(grid,),
        in_specs=[
            pl.BlockSpec((bn, FIN), lambda i: (i, 0)),
            pl.BlockSpec((FIN, F1), lambda i: (0, 0)),
            pl.BlockSpec((F1, 8), lambda i: (0, 0)),
            pl.BlockSpec((F1, 8), lambda i: (0, 0)),
        ],
        out_specs=[
            pl.BlockSpec((2, bn, HXW), lambda i: (0, i, 0)),
            pl.BlockSpec((bn, 8), lambda i: (i, 0)),
            pl.BlockSpec((8, 8), lambda i: (0, 0)),
            pl.BlockSpec((8, 8), lambda i: (0, 0)),
        ],
        out_shape=[
            jax.ShapeDtypeStruct((2, N, HXW), jnp.float32),
            jax.ShapeDtypeStruct((N, 8), jnp.float32),
            jax.ShapeDtypeStruct((8, 8), jnp.float32),
            jax.ShapeDtypeStruct((8, 8), jnp.float32),
        ],
    )(x, W1, A1s, A1d)


def _combine1_body(p_ref, d_ref, b_ref, w_ref, as_ref, ad_ref,
                   h2_ref, a_s_ref, a_d_ref, ms_ref, md_ref):
    i = pl.program_id(0)
    bn = p_ref.shape[1]
    hpre = jnp.concatenate([p_ref[0], p_ref[1]], axis=1)
    d8 = jnp.concatenate([d_ref[0, :, 0:4], d_ref[1, :, 0:4]], axis=1)
    inv = 1.0 / (d8 + 1e-16)
    invx = jnp.broadcast_to(inv[:, :, None], (bn, 8, HID)).reshape(bn, F1)
    h = jax.nn.relu(hpre * invx + b_ref[...])
    h2 = jnp.dot(h, w_ref[...], preferred_element_type=jnp.float32)
    h2_ref[...] = h2
    a_s = jnp.dot(h2, as_ref[...], preferred_element_type=jnp.float32)
    a_d = jnp.dot(h2, ad_ref[...], preferred_element_type=jnp.float32)
    a_s_ref[...] = a_s
    a_d_ref[...] = a_d
    pms = jnp.broadcast_to(jnp.max(a_s, axis=0, keepdims=True), (8, 8))
    pmd = jnp.broadcast_to(jnp.max(a_d, axis=0, keepdims=True), (8, 8))

    @pl.when(i == 0)
    def _():
        ms_ref[...] = pms
        md_ref[...] = pmd

    @pl.when(i > 0)
    def _():
        ms_ref[...] = jnp.maximum(ms_ref[...], pms)
        md_ref[...] = jnp.maximum(md_ref[...], pmd)


def _combine1(parts, den, b1r, W2, A2s, A2d):
    bn = 1000
    grid = N // bn
    return pl.pallas_call(
        _combine1_body,
        grid=(grid,),
        in_specs=[
            pl.BlockSpec((2, bn, FH), lambda i: (0, i, 0)),
            pl.BlockSpec((2, bn, 8), lambda i: (0, i, 0)),
            pl.BlockSpec((1, F1), lambda i: (0, 0)),
            pl.BlockSpec((F1, HID), lambda i: (0, 0)),
            pl.BlockSpec((HID, 8), lambda i: (0, 0)),
            pl.BlockSpec((HID, 8), lambda i: (0, 0)),
        ],
        out_specs=[
            pl.BlockSpec((bn, HID), lambda i: (i, 0)),
            pl.BlockSpec((bn, 8), lambda i: (i, 0)),
            pl.BlockSpec((bn, 8), lambda i: (i, 0)),
            pl.BlockSpec((8, 8), lambda i: (0, 0)),
            pl.BlockSpec((8, 8), lambda i: (0, 0)),
        ],
        out_shape=[
            jax.ShapeDtypeStruct((N, HID), jnp.float32),
            jax.ShapeDtypeStruct((N, 8), jnp.float32),
            jax.ShapeDtypeStruct((N, 8), jnp.float32),
            jax.ShapeDtypeStruct((8, 8), jnp.float32),
            jax.ShapeDtypeStruct((8, 8), jnp.float32),
        ],
    )(parts, den, b1r, W2, A2s, A2d)


def _final_body(p_ref, d_ref, b_ref, o_ref):
    inv = 1.0 / (d_ref[0, :, 0:1] + d_ref[1, :, 0:1] + 1e-16)
    o_ref[...] = (p_ref[0] + p_ref[1]) * inv + b_ref[...]


def _final(parts, den, b2r):
    bn = 1000
    return pl.pallas_call(
        _final_body,
        grid=(N // bn,),
        in_specs=[
            pl.BlockSpec((2, bn, HID), lambda i: (0, i, 0)),
            pl.BlockSpec((2, bn, 8), lambda i: (0, i, 0)),
            pl.BlockSpec((1, HID), lambda i: (0, 0)),
        ],
        out_specs=pl.BlockSpec((bn, HID), lambda i: (i, 0)),
        out_shape=jax.ShapeDtypeStruct((N, HID), jnp.float32),
    )(parts, den, b2r)


def _sweep1_body(hx_h, ad_h, src_h, dst_h, mvec_h, zden_h, zout_h,
                 den_h, out_h,
                 didx, sidx0, sidx1, sidx2,
                 hxv0, hxv1, hxv2, adv0, adv1, adv2,
                 exv0, exv1, exv2, msg0, msg1, msg2, mv,
                 si0, si1, si2, sg0, sg1, sg2,
                 den_acc, out_acc):
    sidx = (sidx0, sidx1, sidx2)
    hxv = (hxv0, hxv1, hxv2)
    adv = (adv0, adv1, adv2)
    exv = (exv0, exv1, exv2)
    msg = (msg0, msg1, msg2)
    si = (si0, si1, si2)
    sg = (sg0, sg1, sg2)
    c = lax.axis_index("c")
    s = lax.axis_index("s")
    o = jnp.minimum(s * TROWS, N - TROWS)
    pltpu.sync_copy(zden_h, den_acc.at[pl.ds(o, TROWS)])
    pltpu.sync_copy(zout_h, out_acc.at[pl.ds(o, TROWS)])
    pltpu.sync_copy(dst_h.at[s], didx)
    pltpu.sync_copy(mvec_h.at[c], mv)
    plsc.subcore_barrier()
    base_t = s * P1
    c4 = 4 * c

    def issue_idx(g, b):
        pltpu.async_copy(src_h.at[c, s, g], sidx[b], si[b])

    def wait_idx(g, b):
        pltpu.make_async_copy(src_h.at[c, s, g], sidx[b], si[b]).wait()

    def issue_gathers(g, b):
        pltpu.async_copy(hx_h.at[sidx[b]], hxv[b], sg[b])
        pltpu.async_copy(ad_h.at[didx.at[g]], adv[b], sg[b])

    def wait_gathers(g, b):
        pltpu.make_async_copy(hx_h.at[sidx[b]], hxv[b], sg[b]).wait()
        pltpu.make_async_copy(ad_h.at[didx.at[g]], adv[b], sg[b]).wait()

    pltpu.sync_copy(src_h.at[c, s, 0], sidx[0])
    pltpu.sync_copy(src_h.at[c, s, 1], sidx[1])
    issue_gathers(0, 0)
    issue_gathers(1, 1)

    def step(g, b):
        nb = (b + 1) % NBUF
        nnb = (b + 2) % NBUF
        wait_gathers(g, b)
        mvv = mv[...]
        base = base_t + g * KC

        def lanes(k, carry2):
            i = lax.iota(jnp.int32, L)
            erow = 2 * k + lax.shift_right_logical(i, 3)
            col = lax.bitwise_and(i, 7)
            col4 = lax.bitwise_and(i, 3)
            a = plsc.load_gather(hxv[b], [erow, FH + col4])
            d = plsc.load_gather(adv[b], [erow, c4 + col4])
            z = a + d
            alpha = jnp.maximum(z, 0.2 * z)
            ex = jnp.exp(alpha - mvv)
            gidx = base + erow
            keep = jnp.logical_and(gidx < E2, col < 4)
            ex = jnp.where(keep, ex, 0.0)
            plsc.store_scatter(exv[b], [erow, col], ex)
            return carry2

        lax.fori_loop(0, KC // 2, lanes, 0)
        pltpu.sync_copy(exv[b], den_acc.at[didx.at[g]], add=True)

        @pl.when(jnp.logical_and(g >= 1, g < NCHUNK1 - 1))
        def _():
            wait_idx(g + 1, nb)
            issue_gathers(g + 1, nb)

        @pl.when(g < NCHUNK1 - 2)
        def _():
            issue_idx(g + 2, nnb)

        def edge(e, carry2):
            esp = jnp.full((L,), 0, jnp.int32) + e
            for j in range(4):
                cv = plsc.load_gather(
                    exv[b], [esp, jnp.full((L,), j, jnp.int32)])
                hrow = hxv[b][e, pl.ds(j * L, L)]
                msg[b][e, pl.ds(j * L, L)] = hrow * cv
            return carry2

        lax.fori_loop(0, KC, edge, 0)
        pltpu.sync_copy(msg[b], out_acc.at[didx.at[g]], add=True)

    def trip(t, carry):
        for b in range(NBUF):
            step(NBUF * t + b, b)
        return carry

    lax.fori_loop(0, NCHUNK1 // NBUF, trip, 0)
    plsc.subcore_barrier()
    pltpu.sync_copy(den_acc.at[pl.ds(o, TROWS)],
                    den_h.at[c, pl.ds(o, TROWS)])
    pltpu.sync_copy(out_acc.at[pl.ds(o, TROWS)],
                    out_h.at[c, pl.ds(o, TROWS)])


def _edge_sweep1(hx2, a_d, src4, dst4, mvec2, zden, zout):
    return pl.kernel(
        _sweep1_body,
        out_type=[
            jax.ShapeDtypeStruct((NC, N, 8), jnp.float32),
            jax.ShapeDtypeStruct((NC, N, FH), jnp.float32),
        ],
        mesh=_mesh(),
        compiler_params=_SC_PARAMS,
        scratch_types=(
            [pltpu.VMEM((NCHUNK1, KC), jnp.int32)]
            + [pltpu.VMEM((KC,), jnp.int32)] * 3
            + [pltpu.VMEM((KC, HXW), jnp.float32)] * 3
            + [pltpu.VMEM((KC, 8), jnp.float32)] * 3
            + [pltpu.VMEM((KC, 8), jnp.float32)] * 3
            + [pltpu.VMEM((KC, FH), jnp.float32)] * 3
            + [pltpu.VMEM((L,), jnp.float32)]
            + [pltpu.SemaphoreType.DMA] * 6
            + [pltpu.VMEM_SHARED((N, 8), jnp.float32),
               pltpu.VMEM_SHARED((N, FH), jnp.float32)]
        ),
    )(hx2, a_d, src4, dst4, mvec2, zden, zout)


def _sweep2_body(as_h, ad_h, h_h, src_h, dst_h, mvec_h, zden_h, zout_h,
                 den_h, out_h,
                 didx, sidx0, sidx1, sidx2,
                 asv0, asv1, asv2, adv0, adv1, adv2,
                 exv0, exv1, exv2, hv0, hv1, hv2, mv,
                 si0, si1, si2, sg0, sg1, sg2,
                 den_acc, out_acc):
    sidx = (sidx0, sidx1, sidx2)
    asv = (asv0, asv1, asv2)
    adv = (adv0, adv1, adv2)
    exv = (exv0, exv1, exv2)
    hv = (hv0, hv1, hv2)
    si = (si0, si1, si2)
    sg = (sg0, sg1, sg2)
    c = lax.axis_index("c")
    s = lax.axis_index("s")
    wid = c * NS + s
    o = jnp.minimum(s * TROWS, N - TROWS)
    pltpu.sync_copy(zden_h, den_acc.at[pl.ds(o, TROWS)])
    pltpu.sync_copy(zout_h, out_acc.at[pl.ds(o, TROWS)])
    pltpu.sync_copy(dst_h.at[wid], didx)
    pltpu.sync_copy(mvec_h, mv)
    plsc.subcore_barrier()
    base_w = wid * P

    def issue_idx(g, b):
        pltpu.async_copy(src_h.at[wid, g], sidx[b], si[b])

    def wait_idx(g, b):
        pltpu.make_async_copy(src_h.at[wid, g], sidx[b], si[b]).wait()

    def issue_gathers(g, b):
        pltpu.async_copy(as_h.at[sidx[b]], asv[b], sg[b])
        pltpu.async_copy(ad_h.at[didx.at[g]], adv[b], sg[b])
        pltpu.async_copy(h_h.at[sidx[b]], hv[b], sg[b])

    def wait_gathers(g, b):
        pltpu.make_async_copy(as_h.at[sidx[b]], asv[b], sg[b]).wait()
        pltpu.make_async_copy(ad_h.at[didx.at[g]], adv[b], sg[b]).wait()
        pltpu.make_async_copy(h_h.at[sidx[b]], hv[b], sg[b]).wait()

    pltpu.sync_copy(src_h.at[wid, 0], sidx[0])
    pltpu.sync_copy(src_h.at[wid, 1], sidx[1])
    issue_gathers(0, 0)
    issue_gathers(1, 1)

    def step(g, b):
        nb = (b + 1) % NBUF
        nnb = (b + 2) % NBUF
        wait_gathers(g, b)
        mvv = mv[...]
        base = base_w + g * KC

        def lanes(k, carry2):
            i = lax.iota(jnp.int32, L)
            erow = 2 * k + lax.shift_right_logical(i, 3)
            col = lax.bitwise_and(i, 7)
            a = plsc.load_gather(asv[b], [erow, col])
            d = plsc.load_gather(adv[b], [erow, col])
            z = a + d
            alpha = jnp.maximum(z, 0.2 * z)
            ex = jnp.exp(alpha - mvv)
            gidx = base + erow
            ex = jnp.where(gidx < E2, ex, 0.0)
            plsc.store_scatter(exv[b], [erow, col], ex)
            return carry2

        lax.fori_loop(0, KC // 2, lanes, 0)
        pltpu.sync_copy(exv[b], den_acc.at[didx.at[g]], add=True)

        @pl.when(jnp.logical_and(g >= 1, g < NCHUNK - 1))
        def _():
            wait_idx(g + 1, nb)
            issue_gathers(g + 1, nb)

        @pl.when(g < NCHUNK - 2)
        def _():
            issue_idx(g + 2, nnb)

        def edge(e, carry2):
            esp = jnp.full((L,), 0, jnp.int32) + e
            cv = plsc.load_gather(
                exv[b], [esp, jnp.full((L,), 0, jnp.int32)])
            hrow = hv[b][e, pl.ds(0, L)]
            hv[b][e, pl.ds(0, L)] = hrow * cv
            return carry2

        lax.fori_loop(0, KC, edge, 0)
        pltpu.sync_copy(hv[b], out_acc.at[didx.at[g]], add=True)

    def trip(t, carry):
        for b in range(NBUF):
            step(NBUF * t + b, b)
        return carry

    lax.fori_loop(0, NCHUNK // NBUF, trip, 0)
    plsc.subcore_barrier()
    pltpu.sync_copy(den_acc.at[pl.ds(o, TROWS)],
                    den_h.at[c, pl.ds(o, TROWS)])
    pltpu.sync_copy(out_acc.at[pl.ds(o, TROWS)],
                    out_h.at[c, pl.ds(o, TROWS)])


def _edge_sweep2(a_s, a_d, h2, src3, dst3, mvec, zden, zout):
    return pl.kernel(
        _sweep2_body,
        out_type=[
            jax.ShapeDtypeStruct((NC, N, 8), jnp.float32),
            jax.ShapeDtypeStruct((NC, N, HID), jnp.float32),
        ],
        mesh=_mesh(),
        compiler_params=_SC_PARAMS,
        scratch_types=(
            [pltpu.VMEM((NCHUNK, KC), jnp.int32)]
            + [pltpu.VMEM((KC,), jnp.int32)] * 3
            + [pltpu.VMEM((KC, 8), jnp.float32)] * 9
            + [pltpu.VMEM((KC, HID), jnp.float32)] * 3
            + [pltpu.VMEM((L,), jnp.float32)]
            + [pltpu.SemaphoreType.DMA] * 6
            + [pltpu.VMEM_SHARED((N, 8), jnp.float32),
               pltpu.VMEM_SHARED((N, HID), jnp.float32)]
        ),
    )(a_s, a_d, h2, src3, dst3, mvec, zden, zout)


def kernel(x, edge_index, W1, att_src1, att_dst1, b1, W2, att_src2,
           att_dst2, b2):
    loop = jnp.arange(N, dtype=edge_index.dtype)
    src = jnp.concatenate([edge_index[0], loop,
                           jnp.zeros((E2P - E2,), edge_index.dtype)])
    dst = jnp.concatenate([edge_index[1], loop,
                           jnp.zeros((E2P - E2,), edge_index.dtype)])
    srcT = src.reshape(NS, NCHUNK1, KC)
    src4 = jnp.stack([srcT, srcT + N])
    dst4 = dst.reshape(NS, NCHUNK1, KC)
    src3 = src.reshape(NW, NCHUNK, KC)
    dst3 = dst.reshape(NW, NCHUNK, KC)

    eye8 = jnp.eye(H1, dtype=jnp.float32)
    A1s = (att_src1[:, :, None] * eye8[:, None, :]).reshape(F1, H1)
    A1d = (att_dst1[:, :, None] * eye8[:, None, :]).reshape(F1, H1)
    A2s = jnp.tile(att_src2.reshape(HID, 1), (1, 8))
    A2d = jnp.tile(att_dst2.reshape(HID, 1), (1, 8))

    zout64 = jnp.zeros((TROWS, FH), jnp.float32)
    zden8 = jnp.zeros((TROWS, 8), jnp.float32)
    zout16 = jnp.zeros((TROWS, HID), jnp.float32)

    hx, a_d1, ms1, md1 = _feat1(x, W1, A1s, A1d)
    hx2 = hx.reshape(2 * N, HXW)
    z1 = ms1[0] + md1[0]
    m1 = jnp.maximum(z1, 0.2 * z1)
    m1vec2 = jnp.stack([jnp.tile(m1[0:4], 4), jnp.tile(m1[4:8], 4)])
    den1, out1p = _edge_sweep1(hx2, a_d1, src4, dst4, m1vec2,
                               zden8, zout64)

    h2, a_s2, a_d2, ms2, md2 = _combine1(out1p, den1, b1.reshape(1, F1),
                                         W2, A2s, A2d)
    z2 = ms2[0] + md2[0]
    m2vec = jnp.tile(jnp.maximum(z2, 0.2 * z2), 2)
    den2, out2p = _edge_sweep2(a_s2, a_d2, h2, src3, dst3, m2vec,
                               zden8, zout16)

    return _final(out2p, den2, b2.reshape(1, HID))


# speedup vs baseline: 51.7691x; 1.0707x over previous
import functools

import jax
import jax.numpy as jnp
from jax import lax
from jax.experimental import pallas as pl
from jax.experimental.pallas import tpu as pltpu
from jax.experimental.pallas import tpu_sc as plsc

N = 10000
E = 320000
E2 = E + N
FIN = 128
H1 = 8
HID = 16
F1 = H1 * HID
FH = F1 // 2
HXW = FH + 8

NC = 2
NS = 16
NW = NC * NS
L = 16

KC = 128
NCHUNK = 81
P = KC * NCHUNK
NCHUNK1 = 162
P1 = KC * NCHUNK1
E2P = P * NW
NBUF = 3
TROWS = 632


@functools.cache
def _mesh():
    return plsc.VectorSubcoreMesh(
        core_axis_name="c", subcore_axis_name="s",
        num_cores=NC, num_subcores=NS)


_SC_PARAMS = pltpu.CompilerParams(
    use_tc_tiling_on_sc=False, needs_layout_passes=False)


def _feat_body(x_ref, w_ref, as_ref, ad_ref, hx_ref, a_d_ref, ms_ref,
               md_ref):
    i = pl.program_id(0)
    bn = x_ref.shape[0]
    h = jnp.dot(x_ref[...], w_ref[...], preferred_element_type=jnp.float32)
    a_s = jnp.dot(h, as_ref[...], preferred_element_type=jnp.float32)
    a_d = jnp.dot(h, ad_ref[...], preferred_element_type=jnp.float32)
    a_d_ref[...] = a_d
    pad = jnp.zeros((bn, 4), jnp.float32)
    hx_ref[0] = jnp.concatenate([h[:, :FH], a_s[:, :4], pad], axis=1)
    hx_ref[1] = jnp.concatenate([h[:, FH:], a_s[:, 4:], pad], axis=1)
    pms = jnp.broadcast_to(jnp.max(a_s, axis=0, keepdims=True), (8, 8))
    pmd = jnp.broadcast_to(jnp.max(a_d, axis=0, keepdims=True), (8, 8))

    @pl.when(i == 0)
    def _():
        ms_ref[...] = pms
        md_ref[...] = pmd

    @pl.when(i > 0)
    def _():
        ms_ref[...] = jnp.maximum(ms_ref[...], pms)
        md_ref[...] = jnp.maximum(md_ref[...], pmd)


def _feat1(x, W1, A1s, A1d):
    bn = 1000
    grid = N // bn
    return pl.pallas_call(
        _feat_body,
        grid=(grid,),
        in_specs=[
            pl.BlockSpec((bn, FIN), lambda i: (i, 0)),
            pl.BlockSpec((FIN, F1), lambda i: (0, 0)),
            pl.BlockSpec((F1, 8), lambda i: (0, 0)),
            pl.BlockSpec((F1, 8), lambda i: (0, 0)),
        ],
        out_specs=[
            pl.BlockSpec((2, bn, HXW), lambda i: (0, i, 0)),
            pl.BlockSpec((bn, 8), lambda i: (i, 0)),
            pl.BlockSpec((8, 8), lambda i: (0, 0)),
            pl.BlockSpec((8, 8), lambda i: (0, 0)),
        ],
        out_shape=[
            jax.ShapeDtypeStruct((2, N, HXW), jnp.float32),
            jax.ShapeDtypeStruct((N, 8), jnp.float32),
            jax.ShapeDtypeStruct((8, 8), jnp.float32),
            jax.ShapeDtypeStruct((8, 8), jnp.float32),
        ],
    )(x, W1, A1s, A1d)


def _combine1_body(p_ref, d_ref, b_ref, w_ref, as_ref, ad_ref,
                   h2_ref, a_s_ref, a_d_ref, ms_ref, md_ref):
    i = pl.program_id(0)
    bn = p_ref.shape[1]
    hpre = jnp.concatenate([p_ref[0], p_ref[1]], axis=1)
    d8 = jnp.concatenate([d_ref[0, :, 0:4], d_ref[1, :, 0:4]], axis=1)
    inv = 1.0 / (d8 + 1e-16)
    invx = jnp.broadcast_to(inv[:, :, None], (bn, 8, HID)).reshape(bn, F1)
    h = jax.nn.relu(hpre * invx + b_ref[...])
    h2 = jnp.dot(h, w_ref[...], preferred_element_type=jnp.float32)
    h2_ref[...] = h2
    a_s = jnp.dot(h2, as_ref[...], preferred_element_type=jnp.float32)
    a_d = jnp.dot(h2, ad_ref[...], preferred_element_type=jnp.float32)
    a_s_ref[...] = a_s
    a_d_ref[...] = a_d
    pms = jnp.broadcast_to(jnp.max(a_s, axis=0, keepdims=True), (8, 8))
    pmd = jnp.broadcast_to(jnp.max(a_d, axis=0, keepdims=True), (8, 8))

    @pl.when(i == 0)
    def _():
        ms_ref[...] = pms
        md_ref[...] = pmd

    @pl.when(i > 0)
    def _():
        ms_ref[...] = jnp.maximum(ms_ref[...], pms)
        md_ref[...] = jnp.maximum(md_ref[...], pmd)


def _combine1(parts, den, b1r, W2, A2s, A2d):
    bn = 1000
    grid = N // bn
    return pl.pallas_call(
        _combine1_body,
        grid=(grid,),
        in_specs=[
            pl.BlockSpec((2, bn, FH), lambda i: (0, i, 0)),
            pl.BlockSpec((2, bn, 8), lambda i: (0, i, 0)),
            pl.BlockSpec((1, F1), lambda i: (0, 0)),
            pl.BlockSpec((F1, HID), lambda i: (0, 0)),
            pl.BlockSpec((HID, 8), lambda i: (0, 0)),
            pl.BlockSpec((HID, 8), lambda i: (0, 0)),
        ],
        out_specs=[
            pl.BlockSpec((bn, HID), lambda i: (i, 0)),
            pl.BlockSpec((bn, 8), lambda i: (i, 0)),
            pl.BlockSpec((bn, 8), lambda i: (i, 0)),
            pl.BlockSpec((8, 8), lambda i: (0, 0)),
            pl.BlockSpec((8, 8), lambda i: (0, 0)),
        ],
        out_shape=[
            jax.ShapeDtypeStruct((N, HID), jnp.float32),
            jax.ShapeDtypeStruct((N, 8), jnp.float32),
            jax.ShapeDtypeStruct((N, 8), jnp.float32),
            jax.ShapeDtypeStruct((8, 8), jnp.float32),
            jax.ShapeDtypeStruct((8, 8), jnp.float32),
        ],
    )(parts, den, b1r, W2, A2s, A2d)


def _final_body(p_ref, d_ref, b_ref, o_ref):
    inv = 1.0 / (d_ref[0, :, 0:1] + d_ref[1, :, 0:1] + 1e-16)
    o_ref[...] = (p_ref[0] + p_ref[1]) * inv + b_ref[...]


def _final(parts, den, b2r):
    bn = 1000
    return pl.pallas_call(
        _final_body,
        grid=(N // bn,),
        in_specs=[
            pl.BlockSpec((2, bn, HID), lambda i: (0, i, 0)),
            pl.BlockSpec((2, bn, 8), lambda i: (0, i, 0)),
            pl.BlockSpec((1, HID), lambda i: (0, 0)),
        ],
        out_specs=pl.BlockSpec((bn, HID), lambda i: (i, 0)),
        out_shape=jax.ShapeDtypeStruct((N, HID), jnp.float32),
    )(parts, den, b2r)


def _sweep1_body(hx_h, ad_h, src_h, dst_h, mvec_h, zden_h, zout_h,
                 den_h, out_h,
                 didx, sidx0, sidx1, sidx2,
                 hxv0, hxv1, hxv2, adv0, adv1, adv2,
                 exv0, exv1, exv2, msg0, msg1, msg2, mv,
                 si0, si1, si2, sg0, sg1, sg2, ss0, ss1, ss2,
                 den_acc, out_acc):
    sidx = (sidx0, sidx1, sidx2)
    hxv = (hxv0, hxv1, hxv2)
    adv = (adv0, adv1, adv2)
    exv = (exv0, exv1, exv2)
    msg = (msg0, msg1, msg2)
    si = (si0, si1, si2)
    sg = (sg0, sg1, sg2)
    ss = (ss0, ss1, ss2)
    c = lax.axis_index("c")
    s = lax.axis_index("s")
    o = jnp.minimum(s * TROWS, N - TROWS)
    pltpu.sync_copy(zden_h, den_acc.at[pl.ds(o, TROWS)])
    pltpu.sync_copy(zout_h, out_acc.at[pl.ds(o, TROWS)])
    pltpu.sync_copy(dst_h.at[s], didx)
    pltpu.sync_copy(mvec_h.at[c], mv)
    plsc.subcore_barrier()
    base_t = s * P1
    c4 = 4 * c

    def issue_idx(g, b):
        pltpu.async_copy(src_h.at[c, s, g], sidx[b], si[b])

    def wait_idx(g, b):
        pltpu.make_async_copy(src_h.at[c, s, g], sidx[b], si[b]).wait()

    def issue_gathers(g, b):
        pltpu.async_copy(hx_h.at[sidx[b]], hxv[b], sg[b])
        pltpu.async_copy(ad_h.at[didx.at[g]], adv[b], sg[b])

    def wait_gathers(g, b):
        pltpu.make_async_copy(hx_h.at[sidx[b]], hxv[b], sg[b]).wait()
        pltpu.make_async_copy(ad_h.at[didx.at[g]], adv[b], sg[b]).wait()

    def wait_scatters(g, b):
        pltpu.make_async_copy(exv[b], den_acc.at[didx.at[g]], ss[b]).wait()
        pltpu.make_async_copy(msg[b], out_acc.at[didx.at[g]], ss[b]).wait()

    pltpu.sync_copy(src_h.at[c, s, 0], sidx[0])
    pltpu.sync_copy(src_h.at[c, s, 1], sidx[1])
    issue_gathers(0, 0)
    issue_gathers(1, 1)

    def step(g, b):
        nb = (b + 1) % NBUF
        nnb = (b + 2) % NBUF
        wait_gathers(g, b)
        mvv = mv[...]
        base = base_t + g * KC

        def lanes(k, carry2):
            i = lax.iota(jnp.int32, L)
            erow = 2 * k + lax.shift_right_logical(i, 3)
            col = lax.bitwise_and(i, 7)
            col4 = lax.bitwise_and(i, 3)
            a = plsc.load_gather(hxv[b], [erow, FH + col4])
            d = plsc.load_gather(adv[b], [erow, c4 + col4])
            z = a + d
            alpha = jnp.maximum(z, 0.2 * z)
            ex = jnp.exp(alpha - mvv)
            gidx = base + erow
            keep = jnp.logical_and(gidx < E2, col < 4)
            ex = jnp.where(keep, ex, 0.0)
            plsc.store_scatter(exv[b], [erow, col], ex)
            return carry2

        lax.fori_loop(0, KC // 2, lanes, 0)
        pltpu.async_copy(exv[b], den_acc.at[didx.at[g]], ss[b], add=True)

        @pl.when(g >= 2)
        def _():
            wait_scatters(g - 2, nb)

        @pl.when(jnp.logical_and(g >= 1, g < NCHUNK1 - 1))
        def _():
            wait_idx(g + 1, nb)
            issue_gathers(g + 1, nb)

        @pl.when(g < NCHUNK1 - 2)
        def _():
            issue_idx(g + 2, nnb)

        def edge(m, carry2):
            for u in range(2):
                e = 2 * m + u
                esp = jnp.full((L,), 0, jnp.int32) + e
                for j in range(4):
                    cv = plsc.load_gather(
                        exv[b], [esp, jnp.full((L,), j, jnp.int32)])
                    hrow = hxv[b][e, pl.ds(j * L, L)]
                    msg[b][e, pl.ds(j * L, L)] = hrow * cv
            return carry2

        lax.fori_loop(0, KC // 2, edge, 0)
        pltpu.async_copy(msg[b], out_acc.at[didx.at[g]], ss[b], add=True)

    def trip(t, carry):
        for b in range(NBUF):
            step(NBUF * t + b, b)
        return carry

    lax.fori_loop(0, NCHUNK1 // NBUF, trip, 0)
    wait_scatters(NCHUNK1 - 2, (NCHUNK1 - 2) % NBUF)
    wait_scatters(NCHUNK1 - 1, (NCHUNK1 - 1) % NBUF)
    plsc.subcore_barrier()
    pltpu.sync_copy(den_acc.at[pl.ds(o, TROWS)],
                    den_h.at[c, pl.ds(o, TROWS)])
    pltpu.sync_copy(out_acc.at[pl.ds(o, TROWS)],
                    out_h.at[c, pl.ds(o, TROWS)])


def _edge_sweep1(hx2, a_d, src4, dst4, mvec2, zden, zout):
    return pl.kernel(
        _sweep1_body,
        out_type=[
            jax.ShapeDtypeStruct((NC, N, 8), jnp.float32),
            jax.ShapeDtypeStruct((NC, N, FH), jnp.float32),
        ],
        mesh=_mesh(),
        compiler_params=_SC_PARAMS,
        scratch_types=(
            [pltpu.VMEM((NCHUNK1, KC), jnp.int32)]
            + [pltpu.VMEM((KC,), jnp.int32)] * 3
            + [pltpu.VMEM((KC, HXW), jnp.float32)] * 3
            + [pltpu.VMEM((KC, 8), jnp.float32)] * 3
            + [pltpu.VMEM((KC, 8), jnp.float32)] * 3
            + [pltpu.VMEM((KC, FH), jnp.float32)] * 3
            + [pltpu.VMEM((L,), jnp.float32)]
            + [pltpu.SemaphoreType.DMA] * 9
            + [pltpu.VMEM_SHARED((N, 8), jnp.float32),
               pltpu.VMEM_SHARED((N, FH), jnp.float32)]
        ),
    )(hx2, a_d, src4, dst4, mvec2, zden, zout)


def _sweep2_body(as_h, ad_h, h_h, src_h, dst_h, mvec_h, zden_h, zout_h,
                 den_h, out_h,
                 didx, sidx0, sidx1, sidx2,
                 asv0, asv1, asv2, adv0, adv1, adv2,
                 exv0, exv1, exv2, hv0, hv1, hv2, mv,
                 si0, si1, si2, sg0, sg1, sg2,
                 den_acc, out_acc):
    sidx = (sidx0, sidx1, sidx2)
    asv = (asv0, asv1, asv2)
    adv = (adv0, adv1, adv2)
    exv = (exv0, exv1, exv2)
    hv = (hv0, hv1, hv2)
    si = (si0, si1, si2)
    sg = (sg0, sg1, sg2)
    c = lax.axis_index("c")
    s = lax.axis_index("s")
    wid = c * NS + s
    o = jnp.minimum(s * TROWS, N - TROWS)
    pltpu.sync_copy(zden_h, den_acc.at[pl.ds(o, TROWS)])
    pltpu.sync_copy(zout_h, out_acc.at[pl.ds(o, TROWS)])
    pltpu.sync_copy(dst_h.at[wid], didx)
    pltpu.sync_copy(mvec_h, mv)
    plsc.subcore_barrier()
    base_w = wid * P

    def issue_idx(g, b):
        pltpu.async_copy(src_h.at[wid, g], sidx[b], si[b])

    def wait_idx(g, b):
        pltpu.make_async_copy(src_h.at[wid, g], sidx[b], si[b]).wait()

    def issue_gathers(g, b):
        pltpu.async_copy(as_h.at[sidx[b]], asv[b], sg[b])
        pltpu.async_copy(ad_h.at[didx.at[g]], adv[b], sg[b])
        pltpu.async_copy(h_h.at[sidx[b]], hv[b], sg[b])

    def wait_gathers(g, b):
        pltpu.make_async_copy(as_h.at[sidx[b]], asv[b], sg[b]).wait()
        pltpu.make_async_copy(ad_h.at[didx.at[g]], adv[b], sg[b]).wait()
        pltpu.make_async_copy(h_h.at[sidx[b]], hv[b], sg[b]).wait()

    pltpu.sync_copy(src_h.at[wid, 0], sidx[0])
    pltpu.sync_copy(src_h.at[wid, 1], sidx[1])
    issue_gathers(0, 0)
    issue_gathers(1, 1)

    def step(g, b):
        nb = (b + 1) % NBUF
        nnb = (b + 2) % NBUF
        wait_gathers(g, b)
        mvv = mv[...]
        base = base_w + g * KC

        def lanes(k, carry2):
            i = lax.iota(jnp.int32, L)
            erow = 2 * k + lax.shift_right_logical(i, 3)
            col = lax.bitwise_and(i, 7)
            a = plsc.load_gather(asv[b], [erow, col])
            d = plsc.load_gather(adv[b], [erow, col])
            z = a + d
            alpha = jnp.maximum(z, 0.2 * z)
            ex = jnp.exp(alpha - mvv)
            gidx = base + erow
            ex = jnp.where(gidx < E2, ex, 0.0)
            plsc.store_scatter(exv[b], [erow, col], ex)
            return carry2

        lax.fori_loop(0, KC // 2, lanes, 0)
        pltpu.sync_copy(exv[b], den_acc.at[didx.at[g]], add=True)

        @pl.when(jnp.logical_and(g >= 1, g < NCHUNK - 1))
        def _():
            wait_idx(g + 1, nb)
            issue_gathers(g + 1, nb)

        @pl.when(g < NCHUNK - 2)
        def _():
            issue_idx(g + 2, nnb)

        def edge(e, carry2):
            esp = jnp.full((L,), 0, jnp.int32) + e
            cv = plsc.load_gather(
                exv[b], [esp, jnp.full((L,), 0, jnp.int32)])
            hrow = hv[b][e, pl.ds(0, L)]
            hv[b][e, pl.ds(0, L)] = hrow * cv
            return carry2

        lax.fori_loop(0, KC, edge, 0)
        pltpu.sync_copy(hv[b], out_acc.at[didx.at[g]], add=True)

    def trip(t, carry):
        for b in range(NBUF):
            step(NBUF * t + b, b)
        return carry

    lax.fori_loop(0, NCHUNK // NBUF, trip, 0)
    plsc.subcore_barrier()
    pltpu.sync_copy(den_acc.at[pl.ds(o, TROWS)],
                    den_h.at[c, pl.ds(o, TROWS)])
    pltpu.sync_copy(out_acc.at[pl.ds(o, TROWS)],
                    out_h.at[c, pl.ds(o, TROWS)])


def _edge_sweep2(a_s, a_d, h2, src3, dst3, mvec, zden, zout):
    return pl.kernel(
        _sweep2_body,
        out_type=[
            jax.ShapeDtypeStruct((NC, N, 8), jnp.float32),
            jax.ShapeDtypeStruct((NC, N, HID), jnp.float32),
        ],
        mesh=_mesh(),
        compiler_params=_SC_PARAMS,
        scratch_types=(
            [pltpu.VMEM((NCHUNK, KC), jnp.int32)]
            + [pltpu.VMEM((KC,), jnp.int32)] * 3
            + [pltpu.VMEM((KC, 8), jnp.float32)] * 9
            + [pltpu.VMEM((KC, HID), jnp.float32)] * 3
            + [pltpu.VMEM((L,), jnp.float32)]
            + [pltpu.SemaphoreType.DMA] * 6
            + [pltpu.VMEM_SHARED((N, 8), jnp.float32),
               pltpu.VMEM_SHARED((N, HID), jnp.float32)]
        ),
    )(a_s, a_d, h2, src3, dst3, mvec, zden, zout)


def kernel(x, edge_index, W1, att_src1, att_dst1, b1, W2, att_src2,
           att_dst2, b2):
    loop = jnp.arange(N, dtype=edge_index.dtype)
    src = jnp.concatenate([edge_index[0], loop,
                           jnp.zeros((E2P - E2,), edge_index.dtype)])
    dst = jnp.concatenate([edge_index[1], loop,
                           jnp.zeros((E2P - E2,), edge_index.dtype)])
    srcT = src.reshape(NS, NCHUNK1, KC)
    src4 = jnp.stack([srcT, srcT + N])
    dst4 = dst.reshape(NS, NCHUNK1, KC)
    src3 = src.reshape(NW, NCHUNK, KC)
    dst3 = dst.reshape(NW, NCHUNK, KC)

    eye8 = jnp.eye(H1, dtype=jnp.float32)
    A1s = (att_src1[:, :, None] * eye8[:, None, :]).reshape(F1, H1)
    A1d = (att_dst1[:, :, None] * eye8[:, None, :]).reshape(F1, H1)
    A2s = jnp.tile(att_src2.reshape(HID, 1), (1, 8))
    A2d = jnp.tile(att_dst2.reshape(HID, 1), (1, 8))

    zout64 = jnp.zeros((TROWS, FH), jnp.float32)
    zden8 = jnp.zeros((TROWS, 8), jnp.float32)
    zout16 = jnp.zeros((TROWS, HID), jnp.float32)

    hx, a_d1, ms1, md1 = _feat1(x, W1, A1s, A1d)
    hx2 = hx.reshape(2 * N, HXW)
    z1 = ms1[0] + md1[0]
    m1 = jnp.maximum(z1, 0.2 * z1)
    m1vec2 = jnp.stack([jnp.tile(m1[0:4], 4), jnp.tile(m1[4:8], 4)])
    den1, out1p = _edge_sweep1(hx2, a_d1, src4, dst4, m1vec2,
                               zden8, zout64)

    h2, a_s2, a_d2, ms2, md2 = _combine1(out1p, den1, b1.reshape(1, F1),
                                         W2, A2s, A2d)
    z2 = ms2[0] + md2[0]
    m2vec = jnp.tile(jnp.maximum(z2, 0.2 * z2), 2)
    den2, out2p = _edge_sweep2(a_s2, a_d2, h2, src3, dst3, m2vec,
                               zden8, zout16)

    return _final(out2p, den2, b2.reshape(1, HID))


# speedup vs baseline: 52.0777x; 1.0060x over previous
import functools

import jax
import jax.numpy as jnp
from jax import lax
from jax.experimental import pallas as pl
from jax.experimental.pallas import tpu as pltpu
from jax.experimental.pallas import tpu_sc as plsc

N = 10000
E = 320000
E2 = E + N
FIN = 128
H1 = 8
HID = 16
F1 = H1 * HID
FH = F1 // 2
HXW = FH + 8

NC = 2
NS = 16
NW = NC * NS
L = 16

KC = 128
NCHUNK = 81
P = KC * NCHUNK
NCHUNK1 = 162
P1 = KC * NCHUNK1
E2P = P * NW
NBUF = 3
TROWS = 632


@functools.cache
def _mesh():
    return plsc.VectorSubcoreMesh(
        core_axis_name="c", subcore_axis_name="s",
        num_cores=NC, num_subcores=NS)


_SC_PARAMS = pltpu.CompilerParams(
    use_tc_tiling_on_sc=False, needs_layout_passes=False)


def _feat_body(x_ref, w_ref, as_ref, ad_ref, hx_ref, a_d_ref, ms_ref,
               md_ref):
    i = pl.program_id(0)
    bn = x_ref.shape[0]
    h = jnp.dot(x_ref[...], w_ref[...], preferred_element_type=jnp.float32)
    a_s = jnp.dot(h, as_ref[...], preferred_element_type=jnp.float32)
    a_d = jnp.dot(h, ad_ref[...], preferred_element_type=jnp.float32)
    a_d_ref[...] = a_d
    pad = jnp.zeros((bn, 4), jnp.float32)
    hx_ref[0] = jnp.concatenate([h[:, :FH], a_s[:, :4], pad], axis=1)
    hx_ref[1] = jnp.concatenate([h[:, FH:], a_s[:, 4:], pad], axis=1)
    pms = jnp.broadcast_to(jnp.max(a_s, axis=0, keepdims=True), (8, 8))
    pmd = jnp.broadcast_to(jnp.max(a_d, axis=0, keepdims=True), (8, 8))

    @pl.when(i == 0)
    def _():
        ms_ref[...] = pms
        md_ref[...] = pmd

    @pl.when(i > 0)
    def _():
        ms_ref[...] = jnp.maximum(ms_ref[...], pms)
        md_ref[...] = jnp.maximum(md_ref[...], pmd)


def _feat1(x, W1, A1s, A1d):
    bn = 1000
    grid = N // bn
    return pl.pallas_call(
        _feat_body,
        grid=(grid,),
        in_specs=[
            pl.BlockSpec((bn, FIN), lambda i: (i, 0)),
            pl.BlockSpec((FIN, F1), lambda i: (0, 0)),
            pl.BlockSpec((F1, 8), lambda i: (0, 0)),
            pl.BlockSpec((F1, 8), lambda i: (0, 0)),
        ],
        out_specs=[
            pl.BlockSpec((2, bn, HXW), lambda i: (0, i, 0)),
            pl.BlockSpec((bn, 8), lambda i: (i, 0)),
            pl.BlockSpec((8, 8), lambda i: (0, 0)),
            pl.BlockSpec((8, 8), lambda i: (0, 0)),
        ],
        out_shape=[
            jax.ShapeDtypeStruct((2, N, HXW), jnp.float32),
            jax.ShapeDtypeStruct((N, 8), jnp.float32),
            jax.ShapeDtypeStruct((8, 8), jnp.float32),
            jax.ShapeDtypeStruct((8, 8), jnp.float32),
        ],
    )(x, W1, A1s, A1d)


def _combine1_body(p_ref, d_ref, b_ref, w_ref, as_ref, ad_ref,
                   h2_ref, a_s_ref, a_d_ref, ms_ref, md_ref):
    i = pl.program_id(0)
    bn = p_ref.shape[1]
    hpre = jnp.concatenate([p_ref[0], p_ref[1]], axis=1)
    d8 = jnp.concatenate([d_ref[0, :, 0:4], d_ref[1, :, 0:4]], axis=1)
    inv = 1.0 / (d8 + 1e-16)
    invx = jnp.broadcast_to(inv[:, :, None], (bn, 8, HID)).reshape(bn, F1)
    h = jax.nn.relu(hpre * invx + b_ref[...])
    h2 = jnp.dot(h, w_ref[...], preferred_element_type=jnp.float32)
    h2_ref[...] = h2
    a_s = jnp.dot(h2, as_ref[...], preferred_element_type=jnp.float32)
    a_d = jnp.dot(h2, ad_ref[...], preferred_element_type=jnp.float32)
    a_s_ref[...] = a_s
    a_d_ref[...] = a_d
    pms = jnp.broadcast_to(jnp.max(a_s, axis=0, keepdims=True), (8, 8))
    pmd = jnp.broadcast_to(jnp.max(a_d, axis=0, keepdims=True), (8, 8))

    @pl.when(i == 0)
    def _():
        ms_ref[...] = pms
        md_ref[...] = pmd

    @pl.when(i > 0)
    def _():
        ms_ref[...] = jnp.maximum(ms_ref[...], pms)
        md_ref[...] = jnp.maximum(md_ref[...], pmd)


def _combine1(parts, den, b1r, W2, A2s, A2d):
    bn = 1000
    grid = N // bn
    return pl.pallas_call(
        _combine1_body,
        grid=(grid,),
        in_specs=[
            pl.BlockSpec((2, bn, FH), lambda i: (0, i, 0)),
            pl.BlockSpec((2, bn, 8), lambda i: (0, i, 0)),
            pl.BlockSpec((1, F1), lambda i: (0, 0)),
            pl.BlockSpec((F1, HID), lambda i: (0, 0)),
            pl.BlockSpec((HID, 8), lambda i: (0, 0)),
            pl.BlockSpec((HID, 8), lambda i: (0, 0)),
        ],
        out_specs=[
            pl.BlockSpec((bn, HID), lambda i: (i, 0)),
            pl.BlockSpec((bn, 8), lambda i: (i, 0)),
            pl.BlockSpec((bn, 8), lambda i: (i, 0)),
            pl.BlockSpec((8, 8), lambda i: (0, 0)),
            pl.BlockSpec((8, 8), lambda i: (0, 0)),
        ],
        out_shape=[
            jax.ShapeDtypeStruct((N, HID), jnp.float32),
            jax.ShapeDtypeStruct((N, 8), jnp.float32),
            jax.ShapeDtypeStruct((N, 8), jnp.float32),
            jax.ShapeDtypeStruct((8, 8), jnp.float32),
            jax.ShapeDtypeStruct((8, 8), jnp.float32),
        ],
    )(parts, den, b1r, W2, A2s, A2d)


def _final_body(p_ref, d_ref, b_ref, o_ref):
    inv = 1.0 / (d_ref[0, :, 0:1] + d_ref[1, :, 0:1] + 1e-16)
    o_ref[...] = (p_ref[0] + p_ref[1]) * inv + b_ref[...]


def _final(parts, den, b2r):
    bn = 1000
    return pl.pallas_call(
        _final_body,
        grid=(N // bn,),
        in_specs=[
            pl.BlockSpec((2, bn, HID), lambda i: (0, i, 0)),
            pl.BlockSpec((2, bn, 8), lambda i: (0, i, 0)),
            pl.BlockSpec((1, HID), lambda i: (0, 0)),
        ],
        out_specs=pl.BlockSpec((bn, HID), lambda i: (i, 0)),
        out_shape=jax.ShapeDtypeStruct((N, HID), jnp.float32),
    )(parts, den, b2r)


def _sweep1_body(hx_h, ad_h, src_h, dst_h, mvec_h, zden_h, zout_h,
                 den_h, out_h,
                 didx, sidx0, sidx1, sidx2,
                 hxv0, hxv1, hxv2, adv0, adv1, adv2,
                 exv0, exv1, exv2, msg0, msg1, msg2, mv,
                 si0, si1, si2, sg0, sg1, sg2, ss0, ss1, ss2,
                 den_acc, out_acc):
    sidx = (sidx0, sidx1, sidx2)
    hxv = (hxv0, hxv1, hxv2)
    adv = (adv0, adv1, adv2)
    exv = (exv0, exv1, exv2)
    msg = (msg0, msg1, msg2)
    si = (si0, si1, si2)
    sg = (sg0, sg1, sg2)
    ss = (ss0, ss1, ss2)
    c = lax.axis_index("c")
    s = lax.axis_index("s")
    o = jnp.minimum(s * TROWS, N - TROWS)
    pltpu.sync_copy(zden_h, den_acc.at[pl.ds(o, TROWS)])
    pltpu.sync_copy(zout_h, out_acc.at[pl.ds(o, TROWS)])
    pltpu.sync_copy(dst_h.at[s], didx)
    pltpu.sync_copy(mvec_h.at[c], mv)
    plsc.subcore_barrier()
    base_t = s * P1
    c4 = 4 * c

    def issue_idx(g, b):
        pltpu.async_copy(src_h.at[c, s, g], sidx[b], si[b])

    def wait_idx(g, b):
        pltpu.make_async_copy(src_h.at[c, s, g], sidx[b], si[b]).wait()

    def issue_gathers(g, b):
        pltpu.async_copy(hx_h.at[sidx[b]], hxv[b], sg[b])
        pltpu.async_copy(ad_h.at[didx.at[g]], adv[b], sg[b])

    def wait_gathers(g, b):
        pltpu.make_async_copy(hx_h.at[sidx[b]], hxv[b], sg[b]).wait()
        pltpu.make_async_copy(ad_h.at[didx.at[g]], adv[b], sg[b]).wait()

    def wait_scatters(g, b):
        pltpu.make_async_copy(exv[b], den_acc.at[didx.at[g]], ss[b]).wait()
        pltpu.make_async_copy(msg[b], out_acc.at[didx.at[g]], ss[b]).wait()

    pltpu.sync_copy(src_h.at[c, s, 0], sidx[0])
    pltpu.sync_copy(src_h.at[c, s, 1], sidx[1])
    issue_gathers(0, 0)
    issue_gathers(1, 1)

    def step(g, b):
        nb = (b + 1) % NBUF
        nnb = (b + 2) % NBUF
        wait_gathers(g, b)
        mvv = mv[...]
        base = base_t + g * KC

        def lanes(k, carry2):
            i = lax.iota(jnp.int32, L)
            erow = 2 * k + lax.shift_right_logical(i, 3)
            col = lax.bitwise_and(i, 7)
            col4 = lax.bitwise_and(i, 3)
            a = plsc.load_gather(hxv[b], [erow, FH + col4])
            d = plsc.load_gather(adv[b], [erow, c4 + col4])
            z = a + d
            alpha = jnp.maximum(z, 0.2 * z)
            ex = jnp.exp(alpha - mvv)
            gidx = base + erow
            keep = jnp.logical_and(gidx < E2, col < 4)
            ex = jnp.where(keep, ex, 0.0)
            plsc.store_scatter(exv[b], [erow, col], ex)
            return carry2

        lax.fori_loop(0, KC // 2, lanes, 0)
        pltpu.async_copy(exv[b], den_acc.at[didx.at[g]], ss[b], add=True)

        @pl.when(g >= 2)
        def _():
            wait_scatters(g - 2, nb)

        @pl.when(jnp.logical_and(g >= 1, g < NCHUNK1 - 1))
        def _():
            wait_idx(g + 1, nb)
            issue_gathers(g + 1, nb)

        @pl.when(g < NCHUNK1 - 2)
        def _():
            issue_idx(g + 2, nnb)

        def edge(m, carry2):
            for u in range(2):
                e = 2 * m + u
                esp = jnp.full((L,), 0, jnp.int32) + e
                for j in range(4):
                    cv = plsc.load_gather(
                        exv[b], [esp, jnp.full((L,), j, jnp.int32)])
                    hrow = hxv[b][e, pl.ds(j * L, L)]
                    msg[b][e, pl.ds(j * L, L)] = hrow * cv
            return carry2

        lax.fori_loop(0, KC // 2, edge, 0)
        pltpu.async_copy(msg[b], out_acc.at[didx.at[g]], ss[b], add=True)

    def trip(t, carry):
        for b in range(NBUF):
            step(NBUF * t + b, b)
        return carry

    lax.fori_loop(0, NCHUNK1 // NBUF, trip, 0)
    wait_scatters(NCHUNK1 - 2, (NCHUNK1 - 2) % NBUF)
    wait_scatters(NCHUNK1 - 1, (NCHUNK1 - 1) % NBUF)
    plsc.subcore_barrier()
    pltpu.sync_copy(den_acc.at[pl.ds(o, TROWS)],
                    den_h.at[c, pl.ds(o, TROWS)])
    pltpu.sync_copy(out_acc.at[pl.ds(o, TROWS)],
                    out_h.at[c, pl.ds(o, TROWS)])


def _edge_sweep1(hx2, a_d, src4, dst4, mvec2, zden, zout):
    return pl.kernel(
        _sweep1_body,
        out_type=[
            jax.ShapeDtypeStruct((NC, N, 8), jnp.float32),
            jax.ShapeDtypeStruct((NC, N, FH), jnp.float32),
        ],
        mesh=_mesh(),
        compiler_params=_SC_PARAMS,
        scratch_types=(
            [pltpu.VMEM((NCHUNK1, KC), jnp.int32)]
            + [pltpu.VMEM((KC,), jnp.int32)] * 3
            + [pltpu.VMEM((KC, HXW), jnp.float32)] * 3
            + [pltpu.VMEM((KC, 8), jnp.float32)] * 3
            + [pltpu.VMEM((KC, 8), jnp.float32)] * 3
            + [pltpu.VMEM((KC, FH), jnp.float32)] * 3
            + [pltpu.VMEM((L,), jnp.float32)]
            + [pltpu.SemaphoreType.DMA] * 9
            + [pltpu.VMEM_SHARED((N, 8), jnp.float32),
               pltpu.VMEM_SHARED((N, FH), jnp.float32)]
        ),
    )(hx2, a_d, src4, dst4, mvec2, zden, zout)


def _sweep2_body(as_h, ad_h, h_h, src_h, dst_h, mvec_h, zden_h, zout_h,
                 den_h, out_h,
                 didx, sidx0, sidx1, sidx2,
                 asv0, asv1, asv2, adv0, adv1, adv2,
                 exv0, exv1, exv2, hv0, hv1, hv2, mv,
                 si0, si1, si2, sg0, sg1, sg2, ss0, ss1, ss2,
                 den_acc, out_acc):
    sidx = (sidx0, sidx1, sidx2)
    asv = (asv0, asv1, asv2)
    adv = (adv0, adv1, adv2)
    exv = (exv0, exv1, exv2)
    hv = (hv0, hv1, hv2)
    si = (si0, si1, si2)
    sg = (sg0, sg1, sg2)
    ss = (ss0, ss1, ss2)
    c = lax.axis_index("c")
    s = lax.axis_index("s")
    wid = c * NS + s
    o = jnp.minimum(s * TROWS, N - TROWS)
    pltpu.sync_copy(zden_h, den_acc.at[pl.ds(o, TROWS)])
    pltpu.sync_copy(zout_h, out_acc.at[pl.ds(o, TROWS)])
    pltpu.sync_copy(dst_h.at[wid], didx)
    pltpu.sync_copy(mvec_h, mv)
    plsc.subcore_barrier()
    base_w = wid * P

    def issue_idx(g, b):
        pltpu.async_copy(src_h.at[wid, g], sidx[b], si[b])

    def wait_idx(g, b):
        pltpu.make_async_copy(src_h.at[wid, g], sidx[b], si[b]).wait()

    def issue_gathers(g, b):
        pltpu.async_copy(as_h.at[sidx[b]], asv[b], sg[b])
        pltpu.async_copy(ad_h.at[didx.at[g]], adv[b], sg[b])
        pltpu.async_copy(h_h.at[sidx[b]], hv[b], sg[b])

    def wait_gathers(g, b):
        pltpu.make_async_copy(as_h.at[sidx[b]], asv[b], sg[b]).wait()
        pltpu.make_async_copy(ad_h.at[didx.at[g]], adv[b], sg[b]).wait()
        pltpu.make_async_copy(h_h.at[sidx[b]], hv[b], sg[b]).wait()

    def wait_scatters(g, b):
        pltpu.make_async_copy(exv[b], den_acc.at[didx.at[g]], ss[b]).wait()
        pltpu.make_async_copy(hv[b], out_acc.at[didx.at[g]], ss[b]).wait()

    pltpu.sync_copy(src_h.at[wid, 0], sidx[0])
    pltpu.sync_copy(src_h.at[wid, 1], sidx[1])
    issue_gathers(0, 0)
    issue_gathers(1, 1)

    def step(g, b):
        nb = (b + 1) % NBUF
        nnb = (b + 2) % NBUF
        wait_gathers(g, b)
        mvv = mv[...]
        base = base_w + g * KC

        def lanes(k, carry2):
            i = lax.iota(jnp.int32, L)
            erow = 2 * k + lax.shift_right_logical(i, 3)
            col = lax.bitwise_and(i, 7)
            a = plsc.load_gather(asv[b], [erow, col])
            d = plsc.load_gather(adv[b], [erow, col])
            z = a + d
            alpha = jnp.maximum(z, 0.2 * z)
            ex = jnp.exp(alpha - mvv)
            gidx = base + erow
            ex = jnp.where(gidx < E2, ex, 0.0)
            plsc.store_scatter(exv[b], [erow, col], ex)
            return carry2

        lax.fori_loop(0, KC // 2, lanes, 0)
        pltpu.async_copy(exv[b], den_acc.at[didx.at[g]], ss[b], add=True)

        @pl.when(g >= 2)
        def _():
            wait_scatters(g - 2, nb)

        @pl.when(jnp.logical_and(g >= 1, g < NCHUNK - 1))
        def _():
            wait_idx(g + 1, nb)
            issue_gathers(g + 1, nb)

        @pl.when(g < NCHUNK - 2)
        def _():
            issue_idx(g + 2, nnb)

        def edge(m, carry2):
            for u in range(2):
                e = 2 * m + u
                esp = jnp.full((L,), 0, jnp.int32) + e
                cv = plsc.load_gather(
                    exv[b], [esp, jnp.full((L,), 0, jnp.int32)])
                hrow = hv[b][e, pl.ds(0, L)]
                hv[b][e, pl.ds(0, L)] = hrow * cv
            return carry2

        lax.fori_loop(0, KC // 2, edge, 0)
        pltpu.async_copy(hv[b], out_acc.at[didx.at[g]], ss[b], add=True)

    def trip(t, carry):
        for b in range(NBUF):
            step(NBUF * t + b, b)
        return carry

    lax.fori_loop(0, NCHUNK // NBUF, trip, 0)
    wait_scatters(NCHUNK - 2, (NCHUNK - 2) % NBUF)
    wait_scatters(NCHUNK - 1, (NCHUNK - 1) % NBUF)
    plsc.subcore_barrier()
    pltpu.sync_copy(den_acc.at[pl.ds(o, TROWS)],
                    den_h.at[c, pl.ds(o, TROWS)])
    pltpu.sync_copy(out_acc.at[pl.ds(o, TROWS)],
                    out_h.at[c, pl.ds(o, TROWS)])


def _edge_sweep2(a_s, a_d, h2, src3, dst3, mvec, zden, zout):
    return pl.kernel(
        _sweep2_body,
        out_type=[
            jax.ShapeDtypeStruct((NC, N, 8), jnp.float32),
            jax.ShapeDtypeStruct((NC, N, HID), jnp.float32),
        ],
        mesh=_mesh(),
        compiler_params=_SC_PARAMS,
        scratch_types=(
            [pltpu.VMEM((NCHUNK, KC), jnp.int32)]
            + [pltpu.VMEM((KC,), jnp.int32)] * 3
            + [pltpu.VMEM((KC, 8), jnp.float32)] * 9
            + [pltpu.VMEM((KC, HID), jnp.float32)] * 3
            + [pltpu.VMEM((L,), jnp.float32)]
            + [pltpu.SemaphoreType.DMA] * 9
            + [pltpu.VMEM_SHARED((N, 8), jnp.float32),
               pltpu.VMEM_SHARED((N, HID), jnp.float32)]
        ),
    )(a_s, a_d, h2, src3, dst3, mvec, zden, zout)


def kernel(x, edge_index, W1, att_src1, att_dst1, b1, W2, att_src2,
           att_dst2, b2):
    loop = jnp.arange(N, dtype=edge_index.dtype)
    src = jnp.concatenate([edge_index[0], loop,
                           jnp.zeros((E2P - E2,), edge_index.dtype)])
    dst = jnp.concatenate([edge_index[1], loop,
                           jnp.zeros((E2P - E2,), edge_index.dtype)])
    srcT = src.reshape(NS, NCHUNK1, KC)
    src4 = jnp.stack([srcT, srcT + N])
    dst4 = dst.reshape(NS, NCHUNK1, KC)
    src3 = src.reshape(NW, NCHUNK, KC)
    dst3 = dst.reshape(NW, NCHUNK, KC)

    eye8 = jnp.eye(H1, dtype=jnp.float32)
    A1s = (att_src1[:, :, None] * eye8[:, None, :]).reshape(F1, H1)
    A1d = (att_dst1[:, :, None] * eye8[:, None, :]).reshape(F1, H1)
    A2s = jnp.tile(att_src2.reshape(HID, 1), (1, 8))
    A2d = jnp.tile(att_dst2.reshape(HID, 1), (1, 8))

    zout64 = jnp.zeros((TROWS, FH), jnp.float32)
    zden8 = jnp.zeros((TROWS, 8), jnp.float32)
    zout16 = jnp.zeros((TROWS, HID), jnp.float32)

    hx, a_d1, ms1, md1 = _feat1(x, W1, A1s, A1d)
    hx2 = hx.reshape(2 * N, HXW)
    z1 = ms1[0] + md1[0]
    m1 = jnp.maximum(z1, 0.2 * z1)
    m1vec2 = jnp.stack([jnp.tile(m1[0:4], 4), jnp.tile(m1[4:8], 4)])
    den1, out1p = _edge_sweep1(hx2, a_d1, src4, dst4, m1vec2,
                               zden8, zout64)

    h2, a_s2, a_d2, ms2, md2 = _combine1(out1p, den1, b1.reshape(1, F1),
                                         W2, A2s, A2d)
    z2 = ms2[0] + md2[0]
    m2vec = jnp.tile(jnp.maximum(z2, 0.2 * z2), 2)
    den2, out2p = _edge_sweep2(a_s2, a_d2, h2, src3, dst3, m2vec,
                               zden8, zout16)

    return _final(out2p, den2, b2.reshape(1, HID))
